# Initial kernel scaffold; baseline (speedup 1.0000x reference)
import jax
import jax.numpy as jnp
from jax.experimental import pallas as pl


def kernel(x, embeddings):
    raise NotImplementedError("write your pallas kernel here")


# speedup vs baseline: 9.2416x; 9.2416x over previous
import functools

import jax
import jax.numpy as jnp
from jax import lax
from jax.experimental import pallas as pl
from jax.experimental.pallas import tpu as pltpu
from jax.experimental.pallas import tpu_sc as plsc

V_DIM = 100000
EMB_DIM = 64
BATCH = 4096
CTX = 50

NUM_CORES = 2
NUM_SUBCORES = 16
NW = NUM_CORES * NUM_SUBCORES
BPW = BATCH // NW
CHUNK = 16
NCHUNK = BPW // CHUNK
ROWS = CHUNK * CTX
LANES = 16
NVREG = EMB_DIM // LANES
SCALE = 1.0 / CTX

_mesh = plsc.VectorSubcoreMesh(core_axis_name="c", subcore_axis_name="s")


@functools.partial(
    pl.kernel,
    out_type=jax.ShapeDtypeStruct((BATCH, EMB_DIM), jnp.float32),
    mesh=_mesh,
    compiler_params=pltpu.CompilerParams(use_tc_tiling_on_sc=False),
    scratch_types=[
        pltpu.VMEM((NCHUNK, ROWS), jnp.int32),
        pltpu.VMEM((2, ROWS, EMB_DIM), jnp.float32),
        pltpu.VMEM((BPW, EMB_DIM), jnp.float32),
        pltpu.SemaphoreType.DMA,
        pltpu.SemaphoreType.DMA,
    ],
)
def _cbow_sc(idx_hbm, table_hbm, out_hbm, idx_v, rows_v, out_v, sem0, sem1):
    wid = lax.axis_index("s") * NUM_CORES + lax.axis_index("c")
    sems = (sem0, sem1)

    pltpu.sync_copy(idx_hbm.at[wid], idx_v)

    def start_gather(ch):
        buf = ch % 2
        return pltpu.async_copy(
            table_hbm.at[idx_v.at[ch]], rows_v.at[buf], sems[buf])

    handles = [start_gather(0)]
    for ch in range(NCHUNK):
        if ch + 1 < NCHUNK:
            handles.append(start_gather(ch + 1))
        handles[ch].wait()
        buf = ch % 2
        rv = rows_v.at[buf]
        for r in range(CHUNK):
            base = r * CTX

            def body(j, acc, base=base, rv=rv):
                row = base + j
                return tuple(
                    acc[k] + rv[row, pl.ds(k * LANES, LANES)]
                    for k in range(NVREG)
                )

            acc0 = tuple(jnp.zeros((LANES,), jnp.float32) for _ in range(NVREG))
            acc = lax.fori_loop(0, CTX, body, acc0)
            orow = ch * CHUNK + r
            for k in range(NVREG):
                out_v[orow, pl.ds(k * LANES, LANES)] = acc[k] * SCALE

    pltpu.sync_copy(out_v, out_hbm.at[pl.ds(wid * BPW, BPW)])


def kernel(x, embeddings):
    idx = x.astype(jnp.int32).reshape(NW, NCHUNK, ROWS)
    return _cbow_sc(idx, embeddings)


# speedup vs baseline: 9.7676x; 1.0569x over previous
import functools

import jax
import jax.numpy as jnp
from jax import lax
from jax.experimental import pallas as pl
from jax.experimental.pallas import tpu as pltpu
from jax.experimental.pallas import tpu_sc as plsc

V_DIM = 100000
EMB_DIM = 64
BATCH = 4096
CTX = 50

NUM_CORES = 2
NUM_SUBCORES = 16
NW = NUM_CORES * NUM_SUBCORES
BPW = BATCH // NW
CHUNK = 16
NCHUNK = BPW // CHUNK
ROWS = CHUNK * CTX
LANES = 16
NVREG = EMB_DIM // LANES
UNROLL = 10
SCALE = 1.0 / CTX

_mesh = plsc.VectorSubcoreMesh(core_axis_name="c", subcore_axis_name="s")


@functools.partial(
    pl.kernel,
    out_type=jax.ShapeDtypeStruct((BATCH, EMB_DIM), jnp.float32),
    mesh=_mesh,
    compiler_params=pltpu.CompilerParams(use_tc_tiling_on_sc=False),
    scratch_types=[
        pltpu.VMEM((NCHUNK, ROWS), jnp.int32),
        pltpu.VMEM((2, ROWS, EMB_DIM), jnp.float32),
        pltpu.VMEM((BPW, EMB_DIM), jnp.float32),
        pltpu.SemaphoreType.DMA,
        pltpu.SemaphoreType.DMA,
    ],
)
def _cbow_sc(idx_hbm, table_hbm, out_hbm, idx_v, rows_v, out_v, sem0, sem1):
    wid = lax.axis_index("s") * NUM_CORES + lax.axis_index("c")
    sems = (sem0, sem1)

    pltpu.sync_copy(idx_hbm.at[wid], idx_v)

    def start_gather(ch):
        buf = ch % 2
        return pltpu.async_copy(
            table_hbm.at[idx_v.at[ch]], rows_v.at[buf], sems[buf])

    handles = [start_gather(0)]
    for ch in range(NCHUNK):
        if ch + 1 < NCHUNK:
            handles.append(start_gather(ch + 1))
        handles[ch].wait()
        buf = ch % 2
        rv = rows_v.at[buf]
        def row_body(r, _, ch=ch, rv=rv):
            base = r * CTX

            def body(t, acc):
                row0 = base + t * UNROLL
                acc = list(acc)
                for u in range(UNROLL):
                    for k in range(NVREG):
                        bank = (u % 2) * NVREG + k
                        acc[bank] = acc[bank] + rv[
                            row0 + u, pl.ds(k * LANES, LANES)]
                return tuple(acc)

            acc0 = tuple(
                jnp.zeros((LANES,), jnp.float32) for _ in range(2 * NVREG))
            acc = lax.fori_loop(0, CTX // UNROLL, body, acc0)
            orow = ch * CHUNK + r
            for k in range(NVREG):
                out_v[orow, pl.ds(k * LANES, LANES)] = (
                    acc[k] + acc[NVREG + k]) * SCALE
            return 0

        lax.fori_loop(0, CHUNK, row_body, 0)

    pltpu.sync_copy(out_v, out_hbm.at[pl.ds(wid * BPW, BPW)])


def kernel(x, embeddings):
    idx = x.astype(jnp.int32).reshape(NW, NCHUNK, ROWS)
    return _cbow_sc(idx, embeddings)
